# baseline (speedup 1.0000x reference)
import functools

import jax
import jax.numpy as jnp
from jax import lax
from jax.experimental import pallas as pl
from jax.experimental.pallas import tpu as pltpu
from jax.experimental.pallas import tpu_sc as plsc

B, S = 4096, 200
D_MODEL, HIDDEN, N_CLASSES = 64, 128, 1
NUM_CORES, NUM_SUBCORES, LANES = 2, 16, 16
NW = NUM_CORES * NUM_SUBCORES
ROWS_PER_TILE = B // NW
TOK_PER_TILE = ROWS_PER_TILE * S
CHUNK = 128
NCHUNKS = TOK_PER_TILE // CHUNK
NBUF = 4
SLOT_STRIDE = NUM_SUBCORES * ROWS_PER_TILE


def _pool_body(x_hbm, table_hbm, dst_hbm, out_hbm, idx_v, dst_v,
               b0, b1, b2, b3, zbuf, acc_sh,
               g0, g1, g2, g3, s0, s1, s2, s3):
    c_ax = lax.axis_index("c")
    s_ax = lax.axis_index("s")
    wid = s_ax * NUM_CORES + c_ax
    base = wid * ROWS_PER_TILE

    zv = jnp.zeros((LANES,), jnp.float32)

    def zrow(i, carry):
        for j in range(4):
            zbuf[i, pl.ds(j * LANES, LANES)] = zv
        return carry

    lax.fori_loop(0, ROWS_PER_TILE, zrow, 0)
    for k in range(NBUF):
        pltpu.sync_copy(
            zbuf,
            acc_sh.at[pl.ds(k * SLOT_STRIDE + s_ax * ROWS_PER_TILE,
                            ROWS_PER_TILE)],
        )

    pltpu.sync_copy(x_hbm.at[pl.ds(wid * NCHUNKS, NCHUNKS)], idx_v)
    pltpu.sync_copy(dst_hbm.at[s_ax], dst_v)

    bufs = (b0, b1, b2, b3)
    gsems = (g0, g1, g2, g3)
    ssems = (s0, s1, s2, s3)

    def g_issue(c, buf, sem):
        pltpu.async_copy(table_hbm.at[idx_v.at[c]], buf, sem)

    def s_issue(c, buf, sem):
        pltpu.async_copy(buf, acc_sh.at[dst_v.at[c]], sem, add=True)

    def wait(buf, sem):
        pltpu.make_async_copy(table_hbm.at[pl.ds(0, CHUNK)], buf, sem).wait()

    for k in range(NBUF):
        g_issue(k, bufs[k], gsems[k])

    def quad(q, carry):
        c0 = NBUF * q
        for k in range(NBUF):
            wait(bufs[k], gsems[k])
            s_issue(c0 + k, bufs[k], ssems[k])
        for k in range(NBUF):
            wait(bufs[k], ssems[k])
            g_issue(jnp.minimum(c0 + k + NBUF, NCHUNKS - 1), bufs[k], gsems[k])
        return carry

    lax.fori_loop(0, NCHUNKS // NBUF, quad, 0)
    for k in range(NBUF):
        wait(bufs[k], gsems[k])

    for k in range(NBUF):
        pltpu.sync_copy(
            acc_sh.at[pl.ds(k * SLOT_STRIDE + s_ax * ROWS_PER_TILE,
                            ROWS_PER_TILE)],
            bufs[k],
        )

    def rrow(i, carry):
        for j in range(4):
            sl = pl.ds(j * LANES, LANES)
            zbuf[i, sl] = (bufs[0][i, sl] + bufs[1][i, sl]) + (
                bufs[2][i, sl] + bufs[3][i, sl]
            )
        return carry

    lax.fori_loop(0, ROWS_PER_TILE, rrow, 0)
    pltpu.sync_copy(zbuf, out_hbm.at[pl.ds(base, ROWS_PER_TILE)])


_pool = functools.partial(
    pl.kernel,
    mesh=plsc.VectorSubcoreMesh(core_axis_name="c", subcore_axis_name="s"),
    out_type=jax.ShapeDtypeStruct((B, D_MODEL), jnp.float32),
    scratch_types=[
        pltpu.VMEM((NCHUNKS, CHUNK), jnp.int32),
        pltpu.VMEM((NCHUNKS, CHUNK), jnp.int32),
        pltpu.VMEM((CHUNK, D_MODEL), jnp.float32),
        pltpu.VMEM((CHUNK, D_MODEL), jnp.float32),
        pltpu.VMEM((CHUNK, D_MODEL), jnp.float32),
        pltpu.VMEM((CHUNK, D_MODEL), jnp.float32),
        pltpu.VMEM((ROWS_PER_TILE, D_MODEL), jnp.float32),
        pltpu.VMEM_SHARED((NBUF * SLOT_STRIDE, D_MODEL), jnp.float32),
        pltpu.SemaphoreType.DMA,
        pltpu.SemaphoreType.DMA,
        pltpu.SemaphoreType.DMA,
        pltpu.SemaphoreType.DMA,
        pltpu.SemaphoreType.DMA,
        pltpu.SemaphoreType.DMA,
        pltpu.SemaphoreType.DMA,
        pltpu.SemaphoreType.DMA,
    ],
    compiler_params=pltpu.CompilerParams(use_tc_tiling_on_sc=False),
)(_pool_body)


def _mlp_body(s_ref, x_ref, t0_ref, w1_ref, b1_ref, w2_ref, b2_ref,
              wh_ref, bh_ref, o_ref):
    zf = jnp.sum((x_ref[...] == 0).astype(jnp.float32), axis=1, keepdims=True)
    denom = jnp.maximum(jnp.float32(S) - zf, 1.0)
    pooled = (s_ref[...] - zf * t0_ref[...]) / denom
    h1 = jnp.dot(pooled, w1_ref[...], preferred_element_type=jnp.float32)
    h1 = jnp.maximum(h1 + b1_ref[...], 0.0)
    h2 = jnp.dot(h1, w2_ref[...], preferred_element_type=jnp.float32)
    h2 = jnp.maximum(h2 + b2_ref[...], 0.0)
    o_ref[...] = jnp.dot(h2, wh_ref[...], preferred_element_type=jnp.float32) + bh_ref[...]


_MLP_BLK = 1024


def _mlp(sums, x, table0, W1, b1, W2, b2, Wh, bh):
    grid = (B // _MLP_BLK,)
    return pl.pallas_call(
        _mlp_body,
        grid=grid,
        in_specs=[
            pl.BlockSpec((_MLP_BLK, D_MODEL), lambda i: (i, 0)),
            pl.BlockSpec((_MLP_BLK, S), lambda i: (i, 0)),
            pl.BlockSpec((1, D_MODEL), lambda i: (0, 0)),
            pl.BlockSpec((D_MODEL, HIDDEN), lambda i: (0, 0)),
            pl.BlockSpec((1, HIDDEN), lambda i: (0, 0)),
            pl.BlockSpec((HIDDEN, HIDDEN), lambda i: (0, 0)),
            pl.BlockSpec((1, HIDDEN), lambda i: (0, 0)),
            pl.BlockSpec((HIDDEN, N_CLASSES), lambda i: (0, 0)),
            pl.BlockSpec((1, N_CLASSES), lambda i: (0, 0)),
        ],
        out_specs=pl.BlockSpec((_MLP_BLK, N_CLASSES), lambda i: (i, 0)),
        out_shape=jax.ShapeDtypeStruct((B, N_CLASSES), jnp.float32),
    )(sums, x, table0, W1, b1.reshape(1, HIDDEN), W2, b2.reshape(1, HIDDEN),
      Wh, bh.reshape(1, N_CLASSES))


@jax.jit
def kernel(x, table, W1, b1, W2, b2, Wh, bh):
    slot = (jnp.arange(TOK_PER_TILE, dtype=jnp.int32) // S).reshape(NCHUNKS, CHUNK)
    slot = slot + (jnp.arange(NCHUNKS, dtype=jnp.int32) % NBUF
                   )[:, None] * SLOT_STRIDE
    dst_map = slot[None] + (jnp.arange(NUM_SUBCORES, dtype=jnp.int32)
                            * ROWS_PER_TILE)[:, None, None]
    sums = _pool(x.reshape(B * S // CHUNK, CHUNK), table, dst_map)
    return _mlp(sums, x, table[0:1], W1, b1, W2, b2, Wh, bh)


# speedup vs baseline: 1.4934x; 1.4934x over previous
import functools

import jax
import jax.numpy as jnp
from jax import lax
from jax.experimental import pallas as pl
from jax.experimental.pallas import tpu as pltpu
from jax.experimental.pallas import tpu_sc as plsc

B, S = 4096, 200
D_MODEL, HIDDEN, N_CLASSES = 64, 128, 1
NUM_CORES, NUM_SUBCORES, LANES = 2, 16, 16
NW = NUM_CORES * NUM_SUBCORES
ROWS_PER_TILE = B // NW


def _pool_body(x_hbm, table_hbm, out_hbm, idx_v, rows_a, rows_b, rows_c,
               rows_d, out_v, sem_a, sem_b, sem_c, sem_d):
    wid = lax.axis_index("s") * NUM_CORES + lax.axis_index("c")
    base = wid * ROWS_PER_TILE

    pltpu.sync_copy(x_hbm.at[pl.ds(base * S, ROWS_PER_TILE * S)], idx_v)

    def issue(row, buf, sem):
        off = row * S
        pltpu.async_copy(
            table_hbm.at[idx_v.at[pl.ds(off, 128)]], buf.at[pl.ds(0, 128)], sem
        )
        pltpu.async_copy(
            table_hbm.at[idx_v.at[pl.ds(off + 128, S - 128)]],
            buf.at[pl.ds(128, S - 128)],
            sem,
        )

    def drain(buf, sem):
        pltpu.make_async_copy(table_hbm.at[pl.ds(0, S)], buf, sem).wait()

    zero_acc = jnp.zeros((LANES,), jnp.float32)

    def accum(buf, i_out):
        def acc_body(t, accs):
            a0, a1, a2, a3 = accs
            r = [
                [buf[4 * t + k, pl.ds(j * LANES, LANES)] for j in range(4)]
                for k in range(4)
            ]
            a0 = a0 + ((r[0][0] + r[1][0]) + (r[2][0] + r[3][0]))
            a1 = a1 + ((r[0][1] + r[1][1]) + (r[2][1] + r[3][1]))
            a2 = a2 + ((r[0][2] + r[1][2]) + (r[2][2] + r[3][2]))
            a3 = a3 + ((r[0][3] + r[1][3]) + (r[2][3] + r[3][3]))
            return (a0, a1, a2, a3)

        acc = lax.fori_loop(0, S // 4, acc_body, (zero_acc,) * 4)
        for j in range(4):
            out_v[i_out, pl.ds(j * LANES, LANES)] = acc[j]

    last = ROWS_PER_TILE - 1
    bufs = (rows_a, rows_b, rows_c, rows_d)
    sems = (sem_a, sem_b, sem_c, sem_d)
    for k in range(4):
        issue(k, bufs[k], sems[k])

    def row_quad(g, carry):
        for k in range(4):
            r = 4 * g + k
            drain(bufs[k], sems[k])
            accum(bufs[k], r)
            issue(jnp.minimum(r + 4, last), bufs[k], sems[k])
        return carry

    lax.fori_loop(0, ROWS_PER_TILE // 4, row_quad, 0)
    for k in range(4):
        drain(bufs[k], sems[k])
    pltpu.sync_copy(out_v, out_hbm.at[pl.ds(base, ROWS_PER_TILE)])


_pool = functools.partial(
    pl.kernel,
    mesh=plsc.VectorSubcoreMesh(core_axis_name="c", subcore_axis_name="s"),
    out_type=jax.ShapeDtypeStruct((B, D_MODEL), jnp.float32),
    scratch_types=[
        pltpu.VMEM((ROWS_PER_TILE * S,), jnp.int32),
        pltpu.VMEM((S, D_MODEL), jnp.float32),
        pltpu.VMEM((S, D_MODEL), jnp.float32),
        pltpu.VMEM((S, D_MODEL), jnp.float32),
        pltpu.VMEM((S, D_MODEL), jnp.float32),
        pltpu.VMEM((ROWS_PER_TILE, D_MODEL), jnp.float32),
        pltpu.SemaphoreType.DMA,
        pltpu.SemaphoreType.DMA,
        pltpu.SemaphoreType.DMA,
        pltpu.SemaphoreType.DMA,
    ],
    compiler_params=pltpu.CompilerParams(use_tc_tiling_on_sc=False),
)(_pool_body)


def _mlp_body(s_ref, x_ref, t0_ref, w1_ref, b1_ref, w2_ref, b2_ref,
              wh_ref, bh_ref, o_ref):
    zf = jnp.sum((x_ref[...] == 0).astype(jnp.float32), axis=1, keepdims=True)
    denom = jnp.maximum(jnp.float32(S) - zf, 1.0)
    pooled = (s_ref[...] - zf * t0_ref[...]) / denom
    h1 = jnp.dot(pooled, w1_ref[...], preferred_element_type=jnp.float32)
    h1 = jnp.maximum(h1 + b1_ref[...], 0.0)
    h2 = jnp.dot(h1, w2_ref[...], preferred_element_type=jnp.float32)
    h2 = jnp.maximum(h2 + b2_ref[...], 0.0)
    o_ref[...] = jnp.dot(h2, wh_ref[...], preferred_element_type=jnp.float32) + bh_ref[...]


_MLP_BLK = 1024


def _mlp(sums, x, table0, W1, b1, W2, b2, Wh, bh):
    grid = (B // _MLP_BLK,)
    return pl.pallas_call(
        _mlp_body,
        grid=grid,
        in_specs=[
            pl.BlockSpec((_MLP_BLK, D_MODEL), lambda i: (i, 0)),
            pl.BlockSpec((_MLP_BLK, S), lambda i: (i, 0)),
            pl.BlockSpec((1, D_MODEL), lambda i: (0, 0)),
            pl.BlockSpec((D_MODEL, HIDDEN), lambda i: (0, 0)),
            pl.BlockSpec((1, HIDDEN), lambda i: (0, 0)),
            pl.BlockSpec((HIDDEN, HIDDEN), lambda i: (0, 0)),
            pl.BlockSpec((1, HIDDEN), lambda i: (0, 0)),
            pl.BlockSpec((HIDDEN, N_CLASSES), lambda i: (0, 0)),
            pl.BlockSpec((1, N_CLASSES), lambda i: (0, 0)),
        ],
        out_specs=pl.BlockSpec((_MLP_BLK, N_CLASSES), lambda i: (i, 0)),
        out_shape=jax.ShapeDtypeStruct((B, N_CLASSES), jnp.float32),
    )(sums, x, table0, W1, b1.reshape(1, HIDDEN), W2, b2.reshape(1, HIDDEN),
      Wh, bh.reshape(1, N_CLASSES))


@jax.jit
def kernel(x, table, W1, b1, W2, b2, Wh, bh):
    sums = _pool(x.reshape(-1), table)
    return _mlp(sums, x, table[0:1], W1, b1, W2, b2, Wh, bh)


# speedup vs baseline: 1.4964x; 1.0020x over previous
import functools

import jax
import jax.numpy as jnp
from jax import lax
from jax.experimental import pallas as pl
from jax.experimental.pallas import tpu as pltpu
from jax.experimental.pallas import tpu_sc as plsc

B, S = 4096, 200
D_MODEL, HIDDEN, N_CLASSES = 64, 128, 1
NUM_CORES, NUM_SUBCORES, LANES = 2, 16, 16
NW = NUM_CORES * NUM_SUBCORES
ROWS_PER_TILE = B // NW


def _pool_body(x_hbm, table_hbm, out_hbm, idx_v, rows_a, rows_b, rows_c,
               rows_d, out_v, sem_a, sem_b, sem_c, sem_d):
    wid = lax.axis_index("s") * NUM_CORES + lax.axis_index("c")
    base = wid * ROWS_PER_TILE

    pltpu.sync_copy(x_hbm.at[pl.ds(base, ROWS_PER_TILE)], idx_v)

    def issue(row, buf, sem):
        pltpu.async_copy(
            table_hbm.at[idx_v.at[row, pl.ds(0, 128)]], buf.at[pl.ds(0, 128)],
            sem,
        )
        pltpu.async_copy(
            table_hbm.at[idx_v.at[row, pl.ds(128, S - 128)]],
            buf.at[pl.ds(128, S - 128)],
            sem,
        )

    def drain(buf, sem):
        pltpu.make_async_copy(table_hbm.at[pl.ds(0, S)], buf, sem).wait()

    zero_acc = jnp.zeros((LANES,), jnp.float32)

    def accum(buf, i_out):
        def acc_body(t, accs):
            a0, a1, a2, a3 = accs
            r = [
                [buf[4 * t + k, pl.ds(j * LANES, LANES)] for j in range(4)]
                for k in range(4)
            ]
            a0 = a0 + ((r[0][0] + r[1][0]) + (r[2][0] + r[3][0]))
            a1 = a1 + ((r[0][1] + r[1][1]) + (r[2][1] + r[3][1]))
            a2 = a2 + ((r[0][2] + r[1][2]) + (r[2][2] + r[3][2]))
            a3 = a3 + ((r[0][3] + r[1][3]) + (r[2][3] + r[3][3]))
            return (a0, a1, a2, a3)

        acc = lax.fori_loop(0, S // 4, acc_body, (zero_acc,) * 4)
        for j in range(4):
            out_v[i_out, pl.ds(j * LANES, LANES)] = acc[j]

    last = ROWS_PER_TILE - 1
    bufs = (rows_a, rows_b, rows_c, rows_d)
    sems = (sem_a, sem_b, sem_c, sem_d)
    for k in range(4):
        issue(k, bufs[k], sems[k])

    def row_quad(g, carry):
        for k in range(4):
            r = 4 * g + k
            drain(bufs[k], sems[k])
            accum(bufs[k], r)
            issue(jnp.minimum(r + 4, last), bufs[k], sems[k])
        return carry

    lax.fori_loop(0, ROWS_PER_TILE // 4, row_quad, 0)
    for k in range(4):
        drain(bufs[k], sems[k])
    pltpu.sync_copy(out_v, out_hbm.at[pl.ds(base, ROWS_PER_TILE)])


_pool = functools.partial(
    pl.kernel,
    mesh=plsc.VectorSubcoreMesh(core_axis_name="c", subcore_axis_name="s"),
    out_type=jax.ShapeDtypeStruct((B, D_MODEL), jnp.float32),
    scratch_types=[
        pltpu.VMEM((ROWS_PER_TILE, S), jnp.int32),
        pltpu.VMEM((S, D_MODEL), jnp.float32),
        pltpu.VMEM((S, D_MODEL), jnp.float32),
        pltpu.VMEM((S, D_MODEL), jnp.float32),
        pltpu.VMEM((S, D_MODEL), jnp.float32),
        pltpu.VMEM((ROWS_PER_TILE, D_MODEL), jnp.float32),
        pltpu.SemaphoreType.DMA,
        pltpu.SemaphoreType.DMA,
        pltpu.SemaphoreType.DMA,
        pltpu.SemaphoreType.DMA,
    ],
    compiler_params=pltpu.CompilerParams(use_tc_tiling_on_sc=False),
)(_pool_body)


def _mlp_body(s_ref, x_ref, t0_ref, w1_ref, b1_ref, w2_ref, b2_ref,
              wh_ref, bh_ref, o_ref):
    zf = jnp.sum((x_ref[...] == 0).astype(jnp.float32), axis=1, keepdims=True)
    denom = jnp.maximum(jnp.float32(S) - zf, 1.0)
    pooled = (s_ref[...] - zf * t0_ref[...]) / denom
    h1 = jnp.dot(pooled, w1_ref[...], preferred_element_type=jnp.float32)
    h1 = jnp.maximum(h1 + b1_ref[...], 0.0)
    h2 = jnp.dot(h1, w2_ref[...], preferred_element_type=jnp.float32)
    h2 = jnp.maximum(h2 + b2_ref[...], 0.0)
    o_ref[...] = jnp.dot(h2, wh_ref[...], preferred_element_type=jnp.float32) + bh_ref[...]


_MLP_BLK = 1024


def _mlp(sums, x, table0, W1, b1, W2, b2, Wh, bh):
    grid = (B // _MLP_BLK,)
    return pl.pallas_call(
        _mlp_body,
        grid=grid,
        in_specs=[
            pl.BlockSpec((_MLP_BLK, D_MODEL), lambda i: (i, 0)),
            pl.BlockSpec((_MLP_BLK, S), lambda i: (i, 0)),
            pl.BlockSpec((1, D_MODEL), lambda i: (0, 0)),
            pl.BlockSpec((D_MODEL, HIDDEN), lambda i: (0, 0)),
            pl.BlockSpec((1, HIDDEN), lambda i: (0, 0)),
            pl.BlockSpec((HIDDEN, HIDDEN), lambda i: (0, 0)),
            pl.BlockSpec((1, HIDDEN), lambda i: (0, 0)),
            pl.BlockSpec((HIDDEN, N_CLASSES), lambda i: (0, 0)),
            pl.BlockSpec((1, N_CLASSES), lambda i: (0, 0)),
        ],
        out_specs=pl.BlockSpec((_MLP_BLK, N_CLASSES), lambda i: (i, 0)),
        out_shape=jax.ShapeDtypeStruct((B, N_CLASSES), jnp.float32),
    )(sums, x, table0, W1, b1.reshape(1, HIDDEN), W2, b2.reshape(1, HIDDEN),
      Wh, bh.reshape(1, N_CLASSES))


@jax.jit
def kernel(x, table, W1, b1, W2, b2, Wh, bh):
    sums = _pool(x, table)
    return _mlp(sums, x, table[0:1], W1, b1, W2, b2, Wh, bh)
